# baseline (speedup 1.0000x reference)
import functools

import jax
import jax.numpy as jnp
from jax import lax
from jax.experimental import pallas as pl
from jax.experimental.pallas import tpu as pltpu
from jax.experimental.pallas import tpu_sc as plsc

B = 16384
D = 64

_NC = 2
_NS = 16
NW = _NC * _NS
BPW = B // NW
CH = 128
NCH = BPW // CH


def _take(x, idx):
    dnums = lax.GatherDimensionNumbers(
        offset_dims=(), collapsed_slice_dims=(0,), start_index_map=(0,))
    return lax.gather(x, idx[:, None], dnums, (1,),
                      mode=lax.GatherScatterMode.PROMISE_IN_BOUNDS)


def _attention(w1, b1p, w2, b2p, wv):
    pltpu.sync_copy(w1, wv.at[0])
    pltpu.sync_copy(b1p, wv.at[1])
    pltpu.sync_copy(w2, wv.at[2])
    pltpu.sync_copy(b2p, wv.at[3])
    lane = lax.iota(jnp.int32, 16)
    ridx = lax.convert_element_type(lax.shift_right_logical(lane, 2), jnp.float32)
    t = wv[0, :] * ridx
    u = t + _take(t, lane ^ 8)
    v = u + _take(u, lane ^ 4)
    h = jnp.maximum(v + wv[1, :], 0.0)
    h = jnp.where(lane < 4, h, 0.0)
    hb = _take(h, lax.shift_right_logical(lane, 2))
    t2 = hb * wv[2, :]
    u2 = t2 + _take(t2, lane ^ 8)
    v2 = u2 + _take(u2, lane ^ 4)
    lg = jnp.where(lane < 4, v2 + wv[3, :], -1e30)
    m = lg
    for sh in (8, 4, 2, 1):
        m = jnp.maximum(m, _take(m, lane ^ sh))
    e = jnp.exp(lg - m)
    s = e
    for sh in (8, 4, 2, 1):
        s = s + _take(s, lane ^ sh)
    att = e / s
    return [_take(att, jnp.full((16,), i, jnp.int32)) for i in range(4)]


def _prefetch_idx(ibufs, feats, base, sem):
    hs = []
    for ib, fb in zip(ibufs, feats):
        for c in range(NCH):
            hs.append(pltpu.async_copy(
                fb.at[pl.ds(base + c * CH, CH)], ib.at[c], sem))
    for h in hs:
        h.wait()


def _pipeline(fire, compute, out_write):
    hg = {0: fire(0)}
    ho = {}
    for c in range(NCH):
        if c + 1 < NCH:
            if c - 1 >= 0:
                ho[c - 1].wait()
            hg[c + 1] = fire(c + 1)
        for cp in hg[c]:
            cp.wait()
        compute(c)
        ho[c] = out_write(c)
    ho[NCH - 2].wait()
    ho[NCH - 1].wait()


def _sc_small(f1, f2, f3, e1, e2, e3, w1, b1p, w2, b2p,
              out, wv, i1, i2, i3,
              ga0, gb0, gc0, ga1, gb1, gc1,
              sem_i, sg0, sg1, so0, so1):
    wid = lax.axis_index("s") * _NC + lax.axis_index("c")
    base = wid * BPW
    _, a1, a2, a3 = _attention(w1, b1p, w2, b2p, wv)

    _prefetch_idx((i1, i2, i3), (f1, f2, f3), base, sem_i)
    gbs = ((ga0, gb0, gc0), (ga1, gb1, gc1))
    sgs = (sg0, sg1)
    sos = (so0, so1)

    def fire(c):
        p = c % 2
        gg = gbs[p]
        return [
            pltpu.async_copy(e1.at[i1.at[c]], gg[0], sgs[p]),
            pltpu.async_copy(e2.at[i2.at[c]], gg[1], sgs[p]),
            pltpu.async_copy(e3.at[i3.at[c]], gg[2], sgs[p]),
        ]

    def compute(c):
        gg = gbs[c % 2]

        def body(r, carry):
            for j in range(D // 16):
                sl = pl.ds(16 * j, 16)
                gg[0][r, sl] = (a1 * gg[0][r, sl] + a2 * gg[1][r, sl]
                                + a3 * gg[2][r, sl])
            return carry

        lax.fori_loop(0, CH, body, 0)

    def out_write(c):
        p = c % 2
        return pltpu.async_copy(
            gbs[p][0], out.at[pl.ds(base + c * CH, CH)], sos[p])

    _pipeline(fire, compute, out_write)


def _sc_mid(f0, e0, partial, w1, b1p, w2, b2p,
            out, wv, i0,
            gb0, pb0, gb1, pb1,
            sem_i, sg0, sg1, so0, so1):
    wid = lax.axis_index("s") * _NC + lax.axis_index("c")
    base = wid * BPW
    a0 = _attention(w1, b1p, w2, b2p, wv)[0]

    _prefetch_idx((i0,), (f0,), base, sem_i)
    bufs = ((gb0, pb0), (gb1, pb1))
    sgs = (sg0, sg1)
    sos = (so0, so1)

    def fire(c):
        p = c % 2
        gg, pb = bufs[p]
        return [
            pltpu.async_copy(e0.at[i0.at[c]], gg, sgs[p]),
            pltpu.async_copy(partial.at[pl.ds(base + c * CH, CH)], pb, sgs[p]),
        ]

    def compute(c):
        gg, pb = bufs[c % 2]

        def body(r, carry):
            for j in range(D // 16):
                sl = pl.ds(16 * j, 16)
                gg[r, sl] = a0 * gg[r, sl] + pb[r, sl]
            return carry

        lax.fori_loop(0, CH, body, 0)

    def out_write(c):
        p = c % 2
        return pltpu.async_copy(
            bufs[p][0], out.at[pl.ds(base + c * CH, CH)], sos[p])

    _pipeline(fire, compute, out_write)


def _sc_final(f0, node, partial2,
              out, i0,
              nb0, pb0, nb1, pb1,
              sem_i, sg0, sg1, so0, so1):
    wid = lax.axis_index("s") * _NC + lax.axis_index("c")
    base = wid * BPW

    _prefetch_idx((i0,), (f0,), base, sem_i)
    bufs = ((nb0, pb0), (nb1, pb1))
    sgs = (sg0, sg1)
    sos = (so0, so1)

    def fire(c):
        p = c % 2
        nb, pb = bufs[p]
        return [
            pltpu.async_copy(node.at[i0.at[c]], nb, sgs[p]),
            pltpu.async_copy(partial2.at[pl.ds(base + c * CH, CH)], pb, sgs[p]),
        ]

    def compute(c):
        nb, pb = bufs[c % 2]

        def body(r, carry):
            for j in range(D // 16):
                sl = pl.ds(16 * j, 16)
                nb[r, sl] = nb[r, sl] + pb[r, sl]
            return carry

        lax.fori_loop(0, CH, body, 0)

    def out_write(c):
        p = c % 2
        return pltpu.async_copy(
            bufs[p][0], out.at[pl.ds(base + c * CH, CH)], sos[p])

    _pipeline(fire, compute, out_write)


_kernel_cache = []


def _build():
    if not _kernel_cache:
        mesh = plsc.VectorSubcoreMesh(core_axis_name="c", subcore_axis_name="s")
        params = pltpu.CompilerParams(use_tc_tiling_on_sc=False)
        k_small = functools.partial(
            pl.kernel, mesh=mesh, compiler_params=params,
            out_type=jax.ShapeDtypeStruct((B, D), jnp.float32),
            scratch_types=[
                pltpu.VMEM((4, 16), jnp.float32),
                pltpu.VMEM((NCH, CH), jnp.int32),
                pltpu.VMEM((NCH, CH), jnp.int32),
                pltpu.VMEM((NCH, CH), jnp.int32),
                pltpu.VMEM((CH, D), jnp.float32),
                pltpu.VMEM((CH, D), jnp.float32),
                pltpu.VMEM((CH, D), jnp.float32),
                pltpu.VMEM((CH, D), jnp.float32),
                pltpu.VMEM((CH, D), jnp.float32),
                pltpu.VMEM((CH, D), jnp.float32),
                pltpu.SemaphoreType.DMA,
                pltpu.SemaphoreType.DMA,
                pltpu.SemaphoreType.DMA,
                pltpu.SemaphoreType.DMA,
                pltpu.SemaphoreType.DMA,
            ],
        )(_sc_small)
        k_mid = functools.partial(
            pl.kernel, mesh=mesh, compiler_params=params,
            out_type=jax.ShapeDtypeStruct((B, D), jnp.float32),
            scratch_types=[
                pltpu.VMEM((4, 16), jnp.float32),
                pltpu.VMEM((NCH, CH), jnp.int32),
                pltpu.VMEM((CH, D), jnp.float32),
                pltpu.VMEM((CH, D), jnp.float32),
                pltpu.VMEM((CH, D), jnp.float32),
                pltpu.VMEM((CH, D), jnp.float32),
                pltpu.SemaphoreType.DMA,
                pltpu.SemaphoreType.DMA,
                pltpu.SemaphoreType.DMA,
                pltpu.SemaphoreType.DMA,
                pltpu.SemaphoreType.DMA,
            ],
        )(_sc_mid)
        k_final = functools.partial(
            pl.kernel, mesh=mesh, compiler_params=params,
            out_type=jax.ShapeDtypeStruct((B, D), jnp.float32),
            scratch_types=[
                pltpu.VMEM((NCH, CH), jnp.int32),
                pltpu.VMEM((CH, D), jnp.float32),
                pltpu.VMEM((CH, D), jnp.float32),
                pltpu.VMEM((CH, D), jnp.float32),
                pltpu.VMEM((CH, D), jnp.float32),
                pltpu.SemaphoreType.DMA,
                pltpu.SemaphoreType.DMA,
                pltpu.SemaphoreType.DMA,
                pltpu.SemaphoreType.DMA,
                pltpu.SemaphoreType.DMA,
            ],
        )(_sc_final)
        _kernel_cache.append((k_small, k_mid, k_final))
    return _kernel_cache[0]


@jax.jit
def kernel(feat_0, feat_1, feat_2, feat_3, node_table,
           emb_0, emb_1, emb_2, emb_3, W1, b1, W2, b2):
    w1f = W1.reshape(16)
    w2f = W2.reshape(16)
    b1p = jnp.pad(b1, (0, 12))
    b2p = jnp.pad(b2, (0, 12))
    k_small, k_mid, k_final = _build()
    partial = k_small(feat_1, feat_2, feat_3, emb_1, emb_2, emb_3,
                      w1f, b1p, w2f, b2p)
    partial2 = k_mid(feat_0, emb_0, partial, w1f, b1p, w2f, b2p)
    return k_final(feat_0, node_table, partial2)


# speedup vs baseline: 1.0031x; 1.0031x over previous
import functools

import jax
import jax.numpy as jnp
from jax import lax
from jax.experimental import pallas as pl
from jax.experimental.pallas import tpu as pltpu
from jax.experimental.pallas import tpu_sc as plsc

B = 16384
D = 64

_NC = 2
_NS = 16
NW = _NC * _NS
BPW = B // NW
CH = 128
NCH = BPW // CH


def _take(x, idx):
    dnums = lax.GatherDimensionNumbers(
        offset_dims=(), collapsed_slice_dims=(0,), start_index_map=(0,))
    return lax.gather(x, idx[:, None], dnums, (1,),
                      mode=lax.GatherScatterMode.PROMISE_IN_BOUNDS)


def _attention(w1, b1p, w2, b2p, wv):
    pltpu.sync_copy(w1, wv.at[0])
    pltpu.sync_copy(b1p, wv.at[1])
    pltpu.sync_copy(w2, wv.at[2])
    pltpu.sync_copy(b2p, wv.at[3])
    lane = lax.iota(jnp.int32, 16)
    ridx = lax.convert_element_type(lax.shift_right_logical(lane, 2), jnp.float32)
    t = wv[0, :] * ridx
    u = t + _take(t, lane ^ 8)
    v = u + _take(u, lane ^ 4)
    h = jnp.maximum(v + wv[1, :], 0.0)
    h = jnp.where(lane < 4, h, 0.0)
    hb = _take(h, lax.shift_right_logical(lane, 2))
    t2 = hb * wv[2, :]
    u2 = t2 + _take(t2, lane ^ 8)
    v2 = u2 + _take(u2, lane ^ 4)
    lg = jnp.where(lane < 4, v2 + wv[3, :], -1e30)
    m = lg
    for sh in (8, 4, 2, 1):
        m = jnp.maximum(m, _take(m, lane ^ sh))
    e = jnp.exp(lg - m)
    s = e
    for sh in (8, 4, 2, 1):
        s = s + _take(s, lane ^ sh)
    att = e / s
    return [_take(att, jnp.full((16,), i, jnp.int32)) for i in range(4)]


def _prefetch_idx(ibufs, feats, base, sem):
    hs = []
    for ib, fb in zip(ibufs, feats):
        for c in range(NCH):
            hs.append(pltpu.async_copy(
                fb.at[pl.ds(base + c * CH, CH)], ib.at[c], sem))
    for h in hs:
        h.wait()


def _pipeline(fire, compute, out_write):
    hg = {0: fire(0)}
    ho = {}
    for c in range(NCH):
        if c + 1 < NCH:
            if c - 1 >= 0:
                ho[c - 1].wait()
            hg[c + 1] = fire(c + 1)
        for cp in hg[c]:
            cp.wait()
        compute(c)
        ho[c] = out_write(c)
    ho[NCH - 2].wait()
    ho[NCH - 1].wait()


def _sc_small(f1, f2, f3, e1, e2, e3, w1, b1p, w2, b2p,
              out, wv, i1, i2, i3,
              ga0, gb0, gc0, ga1, gb1, gc1,
              sem_i, sg0, sg1, so0, so1):
    wid = lax.axis_index("s") * _NC + lax.axis_index("c")
    base = wid * BPW
    _, a1, a2, a3 = _attention(w1, b1p, w2, b2p, wv)

    _prefetch_idx((i1, i2, i3), (f1, f2, f3), base, sem_i)
    gbs = ((ga0, gb0, gc0), (ga1, gb1, gc1))
    sgs = (sg0, sg1)
    sos = (so0, so1)

    def fire(c):
        p = c % 2
        gg = gbs[p]
        return [
            pltpu.async_copy(e1.at[i1.at[c]], gg[0], sgs[p]),
            pltpu.async_copy(e2.at[i2.at[c]], gg[1], sgs[p]),
            pltpu.async_copy(e3.at[i3.at[c]], gg[2], sgs[p]),
        ]

    def compute(c):
        gg = gbs[c % 2]

        def body(r, carry):
            for j in range(D // 16):
                sl = pl.ds(16 * j, 16)
                gg[0][r, sl] = (a1 * gg[0][r, sl] + a2 * gg[1][r, sl]
                                + a3 * gg[2][r, sl])
            return carry

        lax.fori_loop(0, CH, body, 0)

    def out_write(c):
        p = c % 2
        return pltpu.async_copy(
            gbs[p][0], out.at[pl.ds(base + c * CH, CH)], sos[p])

    _pipeline(fire, compute, out_write)


def _sc_big(f0, node, e0, partial, w1, b1p, w2, b2p,
            out, wv, i0,
            nb0, g0, pb0, nb1, g1, pb1,
            sem_i, sg0, sg1, so0, so1):
    wid = lax.axis_index("s") * _NC + lax.axis_index("c")
    base = wid * BPW
    a0 = _attention(w1, b1p, w2, b2p, wv)[0]

    _prefetch_idx((i0,), (f0,), base, sem_i)
    bufs = ((nb0, g0, pb0), (nb1, g1, pb1))
    sgs = (sg0, sg1)
    sos = (so0, so1)

    def fire(c):
        p = c % 2
        nb, gg, pb = bufs[p]
        return [
            pltpu.async_copy(node.at[i0.at[c]], nb, sgs[p]),
            pltpu.async_copy(e0.at[i0.at[c]], gg, sgs[p]),
            pltpu.async_copy(partial.at[pl.ds(base + c * CH, CH)], pb, sgs[p]),
        ]

    def compute(c):
        nb, gg, pb = bufs[c % 2]

        def body(r, carry):
            for j in range(D // 16):
                sl = pl.ds(16 * j, 16)
                nb[r, sl] = nb[r, sl] + a0 * gg[r, sl] + pb[r, sl]
            return carry

        lax.fori_loop(0, CH, body, 0)

    def out_write(c):
        p = c % 2
        return pltpu.async_copy(
            bufs[p][0], out.at[pl.ds(base + c * CH, CH)], sos[p])

    _pipeline(fire, compute, out_write)


_kernel_cache = []


def _build():
    if not _kernel_cache:
        mesh = plsc.VectorSubcoreMesh(core_axis_name="c", subcore_axis_name="s")
        params = pltpu.CompilerParams(use_tc_tiling_on_sc=False)
        k_small = functools.partial(
            pl.kernel, mesh=mesh, compiler_params=params,
            out_type=jax.ShapeDtypeStruct((B, D), jnp.float32),
            scratch_types=[
                pltpu.VMEM((4, 16), jnp.float32),
                pltpu.VMEM((NCH, CH), jnp.int32),
                pltpu.VMEM((NCH, CH), jnp.int32),
                pltpu.VMEM((NCH, CH), jnp.int32),
                pltpu.VMEM((CH, D), jnp.float32),
                pltpu.VMEM((CH, D), jnp.float32),
                pltpu.VMEM((CH, D), jnp.float32),
                pltpu.VMEM((CH, D), jnp.float32),
                pltpu.VMEM((CH, D), jnp.float32),
                pltpu.VMEM((CH, D), jnp.float32),
                pltpu.SemaphoreType.DMA,
                pltpu.SemaphoreType.DMA,
                pltpu.SemaphoreType.DMA,
                pltpu.SemaphoreType.DMA,
                pltpu.SemaphoreType.DMA,
            ],
        )(_sc_small)
        k_big = functools.partial(
            pl.kernel, mesh=mesh, compiler_params=params,
            out_type=jax.ShapeDtypeStruct((B, D), jnp.float32),
            scratch_types=[
                pltpu.VMEM((4, 16), jnp.float32),
                pltpu.VMEM((NCH, CH), jnp.int32),
                pltpu.VMEM((CH, D), jnp.float32),
                pltpu.VMEM((CH, D), jnp.float32),
                pltpu.VMEM((CH, D), jnp.float32),
                pltpu.VMEM((CH, D), jnp.float32),
                pltpu.VMEM((CH, D), jnp.float32),
                pltpu.VMEM((CH, D), jnp.float32),
                pltpu.SemaphoreType.DMA,
                pltpu.SemaphoreType.DMA,
                pltpu.SemaphoreType.DMA,
                pltpu.SemaphoreType.DMA,
                pltpu.SemaphoreType.DMA,
            ],
        )(_sc_big)
        _kernel_cache.append((k_small, k_big))
    return _kernel_cache[0]


@jax.jit
def kernel(feat_0, feat_1, feat_2, feat_3, node_table,
           emb_0, emb_1, emb_2, emb_3, W1, b1, W2, b2):
    w1f = W1.reshape(16)
    w2f = W2.reshape(16)
    b1p = jnp.pad(b1, (0, 12))
    b2p = jnp.pad(b2, (0, 12))
    k_small, k_big = _build()
    partial = k_small(feat_1, feat_2, feat_3, emb_1, emb_2, emb_3,
                      w1f, b1p, w2f, b2p)
    return k_big(feat_0, node_table, emb_0, partial, w1f, b1p, w2f, b2p)
